# baseline (speedup 1.0000x reference)
import functools

import jax
import jax.numpy as jnp
from jax import lax
from jax.experimental import pallas as pl
from jax.experimental.pallas import tpu as pltpu
from jax.experimental.pallas import tpu_sc as plsc

VOCAB = 100000
EMBED_DIM = 200
BATCH = 4096
HIST = 200

ROW_BLOCK = 10240
VOCAB_PAD = 102400
HALF_BLOCK = ROW_BLOCK // 2

NC = 2
NS = 16
L = 16
NW = NC * NS
ROWS_W = BATCH // NW
CROWS = 16
N_CHUNK = ROWS_W // CROWS
COLS = tuple(range(0, HIST - L, L)) + (HIST - L,)


def _matvec_body(wt_ref, ta_ref, tb_ref, b_ref, o_ref):
    wt = wt_ref[...]
    dn = (((1,), (1,)), ((), ()))
    o_ref[pl.ds(0, HALF_BLOCK)] = (
        lax.dot_general(wt, ta_ref[...], dn, preferred_element_type=jnp.float32)
        + b_ref[0]
    )[0]
    o_ref[pl.ds(HALF_BLOCK, HALF_BLOCK)] = (
        lax.dot_general(wt, tb_ref[...], dn, preferred_element_type=jnp.float32)
        + b_ref[0]
    )[0]


def _project_table(table, Wt, b):
    return pl.pallas_call(
        _matvec_body,
        grid=(VOCAB_PAD // ROW_BLOCK,),
        in_specs=[
            pl.BlockSpec((1, EMBED_DIM), lambda i: (0, 0)),
            pl.BlockSpec((HALF_BLOCK, EMBED_DIM), lambda i: (2 * i, 0)),
            pl.BlockSpec((HALF_BLOCK, EMBED_DIM), lambda i: (2 * i + 1, 0)),
            pl.BlockSpec(memory_space=pltpu.SMEM),
        ],
        out_specs=pl.BlockSpec((ROW_BLOCK,), lambda i: (i,)),
        out_shape=jax.ShapeDtypeStruct((VOCAB_PAD,), jnp.float32),
    )(Wt, table, table, b)


_SC_MESH = plsc.VectorSubcoreMesh(core_axis_name="c", subcore_axis_name="s")


@functools.partial(
    pl.kernel,
    mesh=_SC_MESH,
    out_type=jax.ShapeDtypeStruct((BATCH, HIST), jnp.float32),
    compiler_params=pltpu.CompilerParams(
        needs_layout_passes=False, skip_device_barrier=True
    ),
    scratch_types=[
        pltpu.VMEM((VOCAB,), jnp.float32),
        pltpu.VMEM((CROWS, HIST), jnp.int32),
        pltpu.VMEM((CROWS, HIST), jnp.int32),
        pltpu.VMEM((CROWS, HIST), jnp.float32),
        pltpu.VMEM((CROWS, HIST), jnp.float32),
        pltpu.SemaphoreType.DMA,
        pltpu.SemaphoreType.DMA((2,)),
        pltpu.SemaphoreType.DMA((2,)),
    ],
)
def _sc_gather(
    tv_hbm, idx_hbm, out_hbm,
    tv_v, idx_v0, idx_v1, out_v0, out_v1, tv_sem, idx_sem, out_sem,
):
    wid = lax.axis_index("s") * NC + lax.axis_index("c")
    base = pl.multiple_of(wid * ROWS_W, 8)
    idx_bufs = (idx_v0, idx_v1)
    out_bufs = (out_v0, out_v1)

    tv_cp = pltpu.async_copy(tv_hbm.at[pl.ds(0, VOCAB)], tv_v, tv_sem)

    def start_idx(ch):
        r0 = pl.multiple_of(base + ch * CROWS, 8)
        return pltpu.async_copy(
            idx_hbm.at[pl.ds(r0, CROWS)], idx_bufs[ch % 2], idx_sem.at[ch % 2]
        )

    def start_out(ch):
        r0 = pl.multiple_of(base + ch * CROWS, 8)
        return pltpu.async_copy(
            out_bufs[ch % 2], out_hbm.at[pl.ds(r0, CROWS)], out_sem.at[ch % 2]
        )

    idx_cp = [None, None]
    out_cp = [None, None]
    idx_cp[0] = start_idx(0)
    tv_cp.wait()
    for ch in range(N_CHUNK):
        b = ch % 2
        if ch + 1 < N_CHUNK:
            idx_cp[(ch + 1) % 2] = start_idx(ch + 1)
        idx_cp[b].wait()
        if out_cp[b] is not None:
            out_cp[b].wait()
        idx_ref = idx_bufs[b]
        o_ref = out_bufs[b]

        def body(h, carry):
            for dr in range(2):
                r = h * 2 + dr
                for c in COLS:
                    iv = idx_ref[r, pl.ds(c, L)]
                    o_ref[r, pl.ds(c, L)] = plsc.load_gather(tv_v, [iv])
            return carry

        lax.fori_loop(0, CROWS // 2, body, 0)
        out_cp[b] = start_out(ch)
    for cp in out_cp:
        if cp is not None:
            cp.wait()


def kernel(inputs, table, W, b):
    tv = _project_table(table, W.reshape(1, EMBED_DIM), b)
    return _sc_gather(tv, inputs.astype(jnp.int32))


# speedup vs baseline: 1.0094x; 1.0094x over previous
import functools

import jax
import jax.numpy as jnp
from jax import lax
from jax.experimental import pallas as pl
from jax.experimental.pallas import tpu as pltpu
from jax.experimental.pallas import tpu_sc as plsc

VOCAB = 100000
EMBED_DIM = 200
BATCH = 4096
HIST = 200

ROW_BLOCK = 10240
VOCAB_PAD = 102400
HALF_BLOCK = ROW_BLOCK // 2

NC = 2
NS = 16
L = 16
NW = NC * NS
ROWS_W = BATCH // NW
CROWS = 16
N_CHUNK = ROWS_W // CROWS
COLS = tuple(range(0, HIST - L, L)) + (HIST - L,)


def _matvec_body(wt_ref, ta_ref, tb_ref, b_ref, o_ref):
    wt = wt_ref[...]
    dn = (((1,), (1,)), ((), ()))
    o_ref[pl.ds(0, HALF_BLOCK)] = (
        lax.dot_general(wt, ta_ref[...], dn, preferred_element_type=jnp.float32)
        + b_ref[0]
    )[0]
    o_ref[pl.ds(HALF_BLOCK, HALF_BLOCK)] = (
        lax.dot_general(wt, tb_ref[...], dn, preferred_element_type=jnp.float32)
        + b_ref[0]
    )[0]


def _project_table(table, Wt, b):
    return pl.pallas_call(
        _matvec_body,
        grid=(VOCAB_PAD // ROW_BLOCK,),
        in_specs=[
            pl.BlockSpec((1, EMBED_DIM), lambda i: (0, 0)),
            pl.BlockSpec((HALF_BLOCK, EMBED_DIM), lambda i: (2 * i, 0)),
            pl.BlockSpec((HALF_BLOCK, EMBED_DIM), lambda i: (2 * i + 1, 0)),
            pl.BlockSpec(memory_space=pltpu.SMEM),
        ],
        out_specs=pl.BlockSpec((ROW_BLOCK,), lambda i: (i,)),
        out_shape=jax.ShapeDtypeStruct((VOCAB_PAD,), jnp.float32),
    )(Wt, table, table, b)


_SC_MESH = plsc.VectorSubcoreMesh(core_axis_name="c", subcore_axis_name="s")


@functools.partial(
    pl.kernel,
    mesh=_SC_MESH,
    out_type=jax.ShapeDtypeStruct((BATCH, HIST), jnp.float32),
    compiler_params=pltpu.CompilerParams(
        needs_layout_passes=False, skip_device_barrier=True
    ),
    scratch_types=[
        pltpu.VMEM((VOCAB,), jnp.float32),
        pltpu.VMEM((CROWS, HIST), jnp.int32),
        pltpu.VMEM((CROWS, HIST), jnp.int32),
        pltpu.VMEM((CROWS, HIST), jnp.float32),
        pltpu.VMEM((CROWS, HIST), jnp.float32),
        pltpu.SemaphoreType.DMA,
        pltpu.SemaphoreType.DMA((2,)),
        pltpu.SemaphoreType.DMA((2,)),
    ],
)
def _sc_gather(
    tv_hbm, idx_hbm, out_hbm,
    tv_v, idx_v0, idx_v1, out_v0, out_v1, tv_sem, idx_sem, out_sem,
):
    wid = lax.axis_index("s") * NC + lax.axis_index("c")
    base = pl.multiple_of(wid * ROWS_W, 8)
    idx_bufs = (idx_v0, idx_v1)
    out_bufs = (out_v0, out_v1)

    tv_cp = pltpu.async_copy(tv_hbm.at[pl.ds(0, VOCAB)], tv_v, tv_sem)

    def start_idx(ch):
        r0 = pl.multiple_of(base + ch * CROWS, 8)
        return pltpu.async_copy(
            idx_hbm.at[pl.ds(r0, CROWS)], idx_bufs[ch % 2], idx_sem.at[ch % 2]
        )

    def start_out(ch):
        r0 = pl.multiple_of(base + ch * CROWS, 8)
        return pltpu.async_copy(
            out_bufs[ch % 2], out_hbm.at[pl.ds(r0, CROWS)], out_sem.at[ch % 2]
        )

    idx_cp = [None, None]
    out_cp = [None, None]
    idx_cp[0] = start_idx(0)
    tv_cp.wait()
    for ch in range(N_CHUNK):
        b = ch % 2
        if ch + 1 < N_CHUNK:
            idx_cp[(ch + 1) % 2] = start_idx(ch + 1)
        idx_cp[b].wait()
        if out_cp[b] is not None:
            out_cp[b].wait()
        idx_ref = idx_bufs[b]
        o_ref = out_bufs[b]

        @plsc.parallel_loop(0, CROWS, 1, unroll=2)
        def _(r):
            for c in COLS:
                iv = idx_ref[r, pl.ds(c, L)]
                o_ref[r, pl.ds(c, L)] = plsc.load_gather(tv_v, [iv])
        out_cp[b] = start_out(ch)
    for cp in out_cp:
        if cp is not None:
            cp.wait()


def kernel(inputs, table, W, b):
    tv = _project_table(table, W.reshape(1, EMBED_DIM), b)
    return _sc_gather(tv, inputs.astype(jnp.int32))
